# baseline (speedup 1.0000x reference)
import functools

import jax
import jax.numpy as jnp
from jax import lax
from jax.experimental import pallas as pl
from jax.experimental.pallas import tpu as pltpu
from jax.experimental.pallas import tpu_sc as plsc

NUM_FEATURES = 26
VOCAB = 100000
DIM = 32
BATCH = 4096

NUM_ROWS = BATCH * NUM_FEATURES
NUM_WORKERS = 32
ROWS_PER_W = NUM_ROWS // NUM_WORKERS
CHUNK = 128
CHUNKS_PER_W = ROWS_PER_W // CHUNK
LANES = 16


def _emb_kernel(x_hbm, tab_hbm, out_hbm, idx_v, gidx_v, rows_v, sem):
    wid = lax.axis_index("s") * 2 + lax.axis_index("c")
    base = wid * ROWS_PER_W

    pltpu.sync_copy(x_hbm.at[pl.ds(base, ROWS_PER_W)], idx_v)

    def compute_chunk(c, carry):
        for l in range(CHUNK // LANES):
            pos0 = base + c * CHUNK + l * LANES
            pos = pos0 + lax.iota(jnp.int32, LANES)
            feat = pos % NUM_FEATURES
            v = idx_v[pl.ds(c * CHUNK + l * LANES, LANES)]
            gidx_v[c, pl.ds(l * LANES, LANES)] = v + feat * VOCAB
        return carry

    lax.fori_loop(0, CHUNKS_PER_W, compute_chunk, 0, unroll=False)

    def fire(c, carry):
        pltpu.async_copy(
            tab_hbm.at[gidx_v.at[c]],
            rows_v.at[pl.ds(c * CHUNK, CHUNK)],
            sem,
        )
        return carry

    lax.fori_loop(0, CHUNKS_PER_W, fire, 0, unroll=False)

    def drain(c, carry):
        pltpu.make_async_copy(
            tab_hbm.at[gidx_v.at[c]],
            rows_v.at[pl.ds(c * CHUNK, CHUNK)],
            sem,
        ).wait()
        return carry

    lax.fori_loop(0, CHUNKS_PER_W, drain, 0, unroll=False)

    pltpu.sync_copy(rows_v, out_hbm.at[pl.ds(base, ROWS_PER_W)])


@jax.jit
def _emb(x_flat, tab_flat):
    mesh = plsc.VectorSubcoreMesh(core_axis_name="c", subcore_axis_name="s")
    run = functools.partial(
        pl.kernel,
        out_type=jax.ShapeDtypeStruct((NUM_ROWS, DIM), jnp.float32),
        mesh=mesh,
        scratch_types=[
            pltpu.VMEM((ROWS_PER_W,), jnp.int32),
            pltpu.VMEM((CHUNKS_PER_W, CHUNK), jnp.int32),
            pltpu.VMEM((ROWS_PER_W, DIM), jnp.float32),
            pltpu.SemaphoreType.DMA,
        ],
        compiler_params=pltpu.CompilerParams(use_tc_tiling_on_sc=False),
    )(_emb_kernel)
    return run(x_flat, tab_flat)


def kernel(x, tables):
    x_flat = x.astype(jnp.int32).reshape(NUM_ROWS)
    tab_flat = tables.reshape(NUM_FEATURES * VOCAB, DIM)
    out = _emb(x_flat, tab_flat)
    return out.reshape(BATCH, NUM_FEATURES * DIM)


# speedup vs baseline: 1.8080x; 1.8080x over previous
import functools

import jax
import jax.numpy as jnp
from jax import lax
from jax.experimental import pallas as pl
from jax.experimental.pallas import tpu as pltpu
from jax.experimental.pallas import tpu_sc as plsc

NUM_FEATURES = 26
VOCAB = 100000
DIM = 32
BATCH = 4096

NUM_PAIRS = NUM_FEATURES * DIM
NUM_WORKERS = 32
PAIRS_PER_W = NUM_PAIRS // NUM_WORKERS
CHUNK = 128
CHUNKS_PER_PAIR = BATCH // CHUNK
LANES = 16


def _emb_kernel(x_hbm, tab_hbm, out_hbm, xv, gidx_v, rows_v, sem):
    wid = lax.axis_index("s") * 2 + lax.axis_index("c")
    p0 = wid * PAIRS_PER_W

    def do_pair(i, carry):
        p = p0 + i
        f = p // DIM
        base = p * VOCAB

        pltpu.sync_copy(x_hbm.at[pl.ds(f * BATCH, BATCH)], xv)

        def compute_chunk(c, carry2):
            for l in range(CHUNK // LANES):
                v = xv[pl.ds(c * CHUNK + l * LANES, LANES)]
                gidx_v[c, pl.ds(l * LANES, LANES)] = v + base
            return carry2

        lax.fori_loop(0, CHUNKS_PER_PAIR, compute_chunk, 0, unroll=False)

        def fire(c, carry2):
            pltpu.async_copy(
                tab_hbm.at[gidx_v.at[c]],
                rows_v.at[pl.ds(c * CHUNK, CHUNK)],
                sem,
            )
            return carry2

        lax.fori_loop(0, CHUNKS_PER_PAIR, fire, 0, unroll=False)

        def drain(c, carry2):
            pltpu.make_async_copy(
                tab_hbm.at[gidx_v.at[c]],
                rows_v.at[pl.ds(c * CHUNK, CHUNK)],
                sem,
            ).wait()
            return carry2

        lax.fori_loop(0, CHUNKS_PER_PAIR, drain, 0, unroll=False)

        pltpu.sync_copy(rows_v, out_hbm.at[pl.ds(p * BATCH, BATCH)])
        return carry

    lax.fori_loop(0, PAIRS_PER_W, do_pair, 0, unroll=False)


@jax.jit
def _emb(x_flat, tab_flat):
    mesh = plsc.VectorSubcoreMesh(core_axis_name="c", subcore_axis_name="s")
    run = functools.partial(
        pl.kernel,
        out_type=jax.ShapeDtypeStruct((NUM_PAIRS * BATCH,), jnp.float32),
        mesh=mesh,
        scratch_types=[
            pltpu.VMEM((BATCH,), jnp.int32),
            pltpu.VMEM((CHUNKS_PER_PAIR, CHUNK), jnp.int32),
            pltpu.VMEM((BATCH,), jnp.float32),
            pltpu.SemaphoreType.DMA,
        ],
    )(_emb_kernel)
    return run(x_flat, tab_flat)


def kernel(x, tables):
    x_flat = x.astype(jnp.int32).T.reshape(NUM_FEATURES * BATCH)
    tab_flat = jnp.transpose(tables, (0, 2, 1)).reshape(
        NUM_FEATURES * DIM * VOCAB
    )
    out_t = _emb(x_flat, tab_flat)
    return out_t.reshape(NUM_PAIRS, BATCH).T


# speedup vs baseline: 1.9788x; 1.0945x over previous
import functools

import jax
import jax.numpy as jnp
from jax import lax
from jax.experimental import pallas as pl
from jax.experimental.pallas import tpu as pltpu
from jax.experimental.pallas import tpu_sc as plsc

NUM_FEATURES = 26
VOCAB = 100000
DIM = 32
BATCH = 4096

NUM_WORKERS = 32
LANES = 16
NB_FULL = VOCAB // 128
NBUCKETS = NB_FULL + 1
NBPAD = 784
TAIL_V0 = NB_FULL * 128
TAIL_W = VOCAB - TAIL_V0
BLOCKS_PER_W = 25
RING = 64


def _bucket_kernel(x_hbm, sorted_hbm, starts_hbm, xv, cur, sorted_v,
                   starts_v):
    wid = lax.axis_index("s") * 2 + lax.axis_index("c")

    @pl.when(wid < NUM_FEATURES)
    def _():
        f = wid
        pltpu.sync_copy(x_hbm.at[pl.ds(f * BATCH, BATCH)], xv)

        iota = lax.iota(jnp.int32, LANES)
        zeros = jnp.zeros((LANES,), jnp.int32)
        ones = jnp.ones((LANES,), jnp.int32)

        def zero_chunk(j, c):
            cur[pl.ds(j * LANES, LANES)] = zeros
            return c

        lax.fori_loop(0, NBPAD, zero_chunk, 0, unroll=False)

        def hist(i, c):
            v = xv[pl.ds(i * LANES, LANES)]
            addr = (v >> 7) * LANES + iota
            plsc.addupdate_scatter(cur, [addr], ones)
            return c

        lax.fori_loop(0, BATCH // LANES, hist, 0, unroll=False)

        def scan(j, run):
            c = cur[pl.ds(j * LANES, LANES)]
            s = plsc.cumsum(c)
            cur[pl.ds(j * LANES, LANES)] = s - c + run
            return run + jnp.sum(c)

        lax.fori_loop(0, NBPAD, scan, jnp.int32(0), unroll=False)

        def bounds(j, c):
            starts_v[pl.ds(j * LANES, LANES)] = plsc.load_gather(
                cur, [(j * LANES + iota) * LANES]
            )
            return c

        lax.fori_loop(0, NBPAD // LANES, bounds, 0, unroll=False)

        def permute(i, c):
            v = xv[pl.ds(i * LANES, LANES)]
            addr = (v >> 7) * LANES + iota
            pos = plsc.load_gather(cur, [addr])
            plsc.store_scatter(cur, [addr], pos + 1)
            packed = (v << 12) + (i * LANES + iota)
            plsc.store_scatter(sorted_v, [pos], packed)
            return c

        lax.fori_loop(0, BATCH // LANES, permute, 0, unroll=False)

        pltpu.sync_copy(sorted_v, sorted_hbm.at[pl.ds(f * BATCH, BATCH)])
        pltpu.sync_copy(starts_v, starts_hbm.at[pl.ds(f * NBPAD, NBPAD)])


def _sget(ref, idx):
    return ref[pl.ds(idx, LANES)][0]


def _sweep_kernel(sorted_hbm, starts_hbm, tab_hbm, out_hbm, starts_v, sv,
                  blk, blkt, ring, sem_blk, sem_out):
    wid = lax.axis_index("s") * 2 + lax.axis_index("c")
    vb0 = wid * BLOCKS_PER_W
    iota = lax.iota(jnp.int32, LANES)

    pltpu.sync_copy(
        starts_hbm, starts_v.at[pl.ds(0, NUM_FEATURES * NBPAD)]
    )

    def fire_row(args):
        src_lo, src_hi, b, f, fc = args
        slot = fc % RING

        @pl.when(fc >= RING)
        def _():
            pltpu.make_async_copy(
                ring.at[0], out_hbm.at[pl.ds(0, DIM)], sem_out
            ).wait()

        ring[slot, pl.ds(0, LANES)] = src_lo
        ring[slot, pl.ds(LANES, LANES)] = src_hi
        pltpu.async_copy(
            ring.at[slot], out_hbm.at[pl.ds(b * 832 + f * DIM, DIM)], sem_out
        )
        return fc + 1

    def issue_read(f, vb, buf):
        pltpu.async_copy(
            tab_hbm.at[pl.ds(f * DIM, DIM), pl.ds(vb * 128, 128)],
            blk.at[buf],
            sem_blk,
        )

    def wait_read(buf):
        pltpu.make_async_copy(
            tab_hbm.at[pl.ds(0, DIM), pl.ds(0, 128)], blk.at[buf], sem_blk
        ).wait()

    def do_feature(f, fc):
        pltpu.sync_copy(sorted_hbm.at[pl.ds(f * BATCH, BATCH)],
                        sv.at[pl.ds(0, BATCH)])

        @pl.when(vb0 < NB_FULL)
        def _():
            issue_read(f, vb0, 0)

        def do_block(k, fc2):
            vb = vb0 + k
            valid = vb < NB_FULL

            @pl.when(valid)
            def _():
                wait_read(k % 2)

            @pl.when(jnp.logical_and(vb + 1 < NB_FULL, k + 1 < BLOCKS_PER_W))
            def _():
                issue_read(f, vb + 1, (k + 1) % 2)

            s0 = _sget(starts_v, f * NBPAD + vb)
            e0 = _sget(starts_v, f * NBPAD + vb + 1)
            s0 = jnp.where(valid, s0, 0)
            e0 = jnp.where(valid, e0, 0)

            def do_entry(e, fc3):
                packed = _sget(sv, e)
                v = packed >> 12
                b = packed & 4095
                col = v - vb * 128
                lo = plsc.load_gather(
                    blk,
                    [jnp.full((LANES,), k % 2, jnp.int32), iota,
                     jnp.full((LANES,), 0, jnp.int32) + col],
                )
                hi = plsc.load_gather(
                    blk,
                    [jnp.full((LANES,), k % 2, jnp.int32), iota + LANES,
                     jnp.full((LANES,), 0, jnp.int32) + col],
                )
                return fire_row((lo, hi, b, f, fc3))

            return lax.fori_loop(s0, e0, do_entry, fc2, unroll=False)

        return lax.fori_loop(0, BLOCKS_PER_W, do_block, fc, unroll=False)

    fc = lax.fori_loop(0, NUM_FEATURES, do_feature, jnp.int32(0),
                       unroll=False)

    has_tail = wid < NUM_FEATURES
    ft = jnp.minimum(wid, NUM_FEATURES - 1)

    @pl.when(has_tail)
    def _():
        pltpu.sync_copy(sorted_hbm.at[pl.ds(ft * BATCH, BATCH)],
                        sv.at[pl.ds(0, BATCH)])
        pltpu.sync_copy(
            tab_hbm.at[pl.ds(ft * DIM, DIM), pl.ds(TAIL_V0, TAIL_W)], blkt
        )

    s0 = jnp.where(has_tail, _sget(starts_v, ft * NBPAD + NB_FULL), 0)
    e0 = jnp.where(has_tail, _sget(starts_v, ft * NBPAD + NB_FULL + 1), 0)

    def tail_entry(e, fc3):
        packed = _sget(sv, e)
        v = packed >> 12
        b = packed & 4095
        col = v - TAIL_V0
        lo = plsc.load_gather(
            blkt, [iota, jnp.full((LANES,), 0, jnp.int32) + col]
        )
        hi = plsc.load_gather(
            blkt, [iota + LANES, jnp.full((LANES,), 0, jnp.int32) + col]
        )
        return fire_row((lo, hi, b, ft, fc3))

    fc2 = lax.fori_loop(s0, e0, tail_entry, fc, unroll=False)

    remaining = jnp.minimum(fc2, RING)

    def drain(j, c):
        @pl.when(j < remaining)
        def _():
            pltpu.make_async_copy(
                ring.at[0], out_hbm.at[pl.ds(0, DIM)], sem_out
            ).wait()
        return c

    lax.fori_loop(0, RING, drain, 0, unroll=False)


@jax.jit
def _emb(x_flat, tab2):
    mesh = plsc.VectorSubcoreMesh(core_axis_name="c", subcore_axis_name="s")

    bucket = functools.partial(
        pl.kernel,
        out_type=(
            jax.ShapeDtypeStruct((NUM_FEATURES * BATCH,), jnp.int32),
            jax.ShapeDtypeStruct((NUM_FEATURES * NBPAD,), jnp.int32),
        ),
        mesh=mesh,
        scratch_types=[
            pltpu.VMEM((BATCH,), jnp.int32),
            pltpu.VMEM((NBPAD * LANES,), jnp.int32),
            pltpu.VMEM((BATCH,), jnp.int32),
            pltpu.VMEM((NBPAD,), jnp.int32),
        ],
        compiler_params=pltpu.CompilerParams(needs_layout_passes=False),
    )(_bucket_kernel)
    sorted_a, starts_a = bucket(x_flat)

    sweep = functools.partial(
        pl.kernel,
        out_type=jax.ShapeDtypeStruct((BATCH * NUM_FEATURES * DIM,),
                                      jnp.float32),
        mesh=mesh,
        scratch_types=[
            pltpu.VMEM((NUM_FEATURES * NBPAD + LANES,), jnp.int32),
            pltpu.VMEM((BATCH + LANES,), jnp.int32),
            pltpu.VMEM((2, DIM, 128), jnp.float32),
            pltpu.VMEM((DIM, TAIL_W), jnp.float32),
            pltpu.VMEM((RING, DIM), jnp.float32),
            pltpu.SemaphoreType.DMA,
            pltpu.SemaphoreType.DMA,
        ],
        compiler_params=pltpu.CompilerParams(
            use_tc_tiling_on_sc=True, needs_layout_passes=False
        ),
    )(_sweep_kernel)
    return sweep(sorted_a, starts_a, tab2)


def kernel(x, tables):
    x_flat = x.astype(jnp.int32).T.reshape(NUM_FEATURES * BATCH)
    tab2 = jnp.transpose(tables, (0, 2, 1)).reshape(NUM_FEATURES * DIM, VOCAB)
    out = _emb(x_flat, tab2)
    return out.reshape(BATCH, NUM_FEATURES * DIM)


# speedup vs baseline: 5.2874x; 2.6720x over previous
import functools

import jax
import jax.numpy as jnp
from jax import lax
from jax.experimental import pallas as pl
from jax.experimental.pallas import tpu as pltpu
from jax.experimental.pallas import tpu_sc as plsc

NUM_FEATURES = 26
VOCAB = 100000
DIM = 32
BATCH = 4096

NUM_WORKERS = 32
LANES = 16
NB_FULL = VOCAB // 128
NBUCKETS = NB_FULL + 1
NBPAD = 784
TAIL_V0 = NB_FULL * 128
TAIL_W = VOCAB - TAIL_V0
CHUNK_B = 5
CW = CHUNK_B * 128
NCHUNKS = 156
CHUNKS_PER_W = 5
NBUF = 3
TAIL_C0 = NCHUNKS * CW
TAIL_CW = VOCAB - TAIL_C0
RING = 64


def _bucket_kernel(x_hbm, sorted_hbm, starts_hbm, xv, cur, sorted_v,
                   starts_v):
    wid = lax.axis_index("s") * 2 + lax.axis_index("c")

    @pl.when(wid < NUM_FEATURES)
    def _():
        f = wid
        pltpu.sync_copy(x_hbm.at[pl.ds(f * BATCH, BATCH)], xv)

        iota = lax.iota(jnp.int32, LANES)
        zeros = jnp.zeros((LANES,), jnp.int32)
        ones = jnp.ones((LANES,), jnp.int32)

        def zero_chunk(j, c):
            cur[pl.ds(j * LANES, LANES)] = zeros
            return c

        lax.fori_loop(0, NBPAD, zero_chunk, 0, unroll=False)

        def hist(i, c):
            v = xv[pl.ds(i * LANES, LANES)]
            addr = (v >> 7) * LANES + iota
            plsc.addupdate_scatter(cur, [addr], ones)
            return c

        lax.fori_loop(0, BATCH // LANES, hist, 0, unroll=False)

        def scan(j, run):
            c = cur[pl.ds(j * LANES, LANES)]
            s = plsc.cumsum(c)
            cur[pl.ds(j * LANES, LANES)] = s - c + run
            return run + jnp.sum(c)

        lax.fori_loop(0, NBPAD, scan, jnp.int32(0), unroll=False)

        def bounds(j, c):
            starts_v[pl.ds(j * LANES, LANES)] = plsc.load_gather(
                cur, [(j * LANES + iota) * LANES]
            )
            return c

        lax.fori_loop(0, NBPAD // LANES, bounds, 0, unroll=False)

        def permute(i, c):
            v = xv[pl.ds(i * LANES, LANES)]
            addr = (v >> 7) * LANES + iota
            pos = plsc.load_gather(cur, [addr])
            plsc.store_scatter(cur, [addr], pos + 1)
            packed = (v << 12) + (i * LANES + iota)
            plsc.store_scatter(sorted_v, [pos], packed)
            return c

        lax.fori_loop(0, BATCH // LANES, permute, 0, unroll=False)

        pltpu.sync_copy(sorted_v, sorted_hbm.at[pl.ds(f * BATCH, BATCH)])
        pltpu.sync_copy(starts_v, starts_hbm.at[pl.ds(f * NBPAD, NBPAD)])


def _sget(ref, idx):
    return ref[pl.ds(idx, LANES)][0]


def _sweep_kernel(sorted_hbm, starts_hbm, tab_hbm, out_hbm, starts_v, sv,
                  blk, blkt, ring, sem_blk, sem_out, sem_sv):
    wid = lax.axis_index("s") * 2 + lax.axis_index("c")
    iota = lax.iota(jnp.int32, LANES)
    ntasks = NUM_FEATURES * CHUNKS_PER_W

    pltpu.sync_copy(
        starts_hbm, starts_v.at[pl.ds(0, NUM_FEATURES * NBPAD)]
    )

    def fire_row(src_lo, src_hi, b, f, fc):
        slot = fc % RING

        @pl.when(fc >= RING)
        def _():
            pltpu.make_async_copy(
                ring.at[0], out_hbm.at[pl.ds(0, DIM)], sem_out
            ).wait()

        ring[slot, pl.ds(0, LANES)] = src_lo
        ring[slot, pl.ds(LANES, LANES)] = src_hi
        pltpu.async_copy(
            ring.at[slot], out_hbm.at[pl.ds(b * 832 + f * DIM, DIM)], sem_out
        )
        return fc + 1

    def chunk_of(t):
        f = t // CHUNKS_PER_W
        c = t % CHUNKS_PER_W
        ch = wid * CHUNKS_PER_W + c
        return f, ch

    def issue_read(t):
        f, ch = chunk_of(t)

        @pl.when(jnp.logical_and(t < ntasks, ch < NCHUNKS))
        def _():
            pltpu.async_copy(
                tab_hbm.at[pl.ds(f * DIM, DIM), pl.ds(ch * CW, CW)],
                blk.at[t % NBUF],
                sem_blk,
            )

    def issue_sv(f):
        @pl.when(f < NUM_FEATURES)
        def _():
            pltpu.async_copy(
                sorted_hbm.at[pl.ds(f * BATCH, BATCH)],
                sv.at[f % 2, pl.ds(0, BATCH)],
                sem_sv,
            )

    def wait_sv():
        pltpu.make_async_copy(
            sorted_hbm.at[pl.ds(0, BATCH)], sv.at[0, pl.ds(0, BATCH)],
            sem_sv,
        ).wait()

    issue_sv(0)
    issue_sv(1)
    wait_sv()
    for t in range(NBUF):
        issue_read(t)

    def do_task(t, fc):
        f, ch = chunk_of(t)
        valid = ch < NCHUNKS

        @pl.when(jnp.logical_and(t % CHUNKS_PER_W == 0, t > 0))
        def _():
            wait_sv()
            issue_sv(f + 1)

        @pl.when(valid)
        def _():
            pltpu.make_async_copy(
                tab_hbm.at[pl.ds(0, DIM), pl.ds(0, CW)], blk.at[t % NBUF],
                sem_blk,
            ).wait()

        vb_first = ch * CHUNK_B
        s0 = _sget(starts_v, f * NBPAD + vb_first)
        e0 = _sget(starts_v, f * NBPAD + vb_first + CHUNK_B)
        s0 = jnp.where(valid, s0, 0)
        e0 = jnp.where(valid, e0, 0)

        def do_entry(e, fc3):
            packed = sv[f % 2, pl.ds(e, LANES)][0]
            v = packed >> 12
            b = packed & 4095
            col = v - vb_first * 128
            lo = plsc.load_gather(
                blk,
                [jnp.full((LANES,), t % NBUF, jnp.int32), iota,
                 jnp.full((LANES,), 0, jnp.int32) + col],
            )
            hi = plsc.load_gather(
                blk,
                [jnp.full((LANES,), t % NBUF, jnp.int32), iota + LANES,
                 jnp.full((LANES,), 0, jnp.int32) + col],
            )
            return fire_row(lo, hi, b, f, fc3)

        fc2 = lax.fori_loop(s0, e0, do_entry, fc, unroll=False)
        issue_read(t + NBUF)
        return fc2

    fc = lax.fori_loop(0, ntasks, do_task, jnp.int32(0), unroll=False)

    has_tail = wid < NUM_FEATURES
    ft = jnp.minimum(wid, NUM_FEATURES - 1)

    @pl.when(has_tail)
    def _():
        pltpu.sync_copy(sorted_hbm.at[pl.ds(ft * BATCH, BATCH)],
                        sv.at[0, pl.ds(0, BATCH)])
        pltpu.sync_copy(
            tab_hbm.at[pl.ds(ft * DIM, DIM), pl.ds(TAIL_C0, TAIL_CW)], blkt
        )

    s0 = jnp.where(has_tail, _sget(starts_v, ft * NBPAD + NCHUNKS * CHUNK_B),
                   0)
    e0 = jnp.where(has_tail,
                   _sget(starts_v, ft * NBPAD + NCHUNKS * CHUNK_B + 2), 0)

    def tail_entry(e, fc3):
        packed = sv[0, pl.ds(e, LANES)][0]
        v = packed >> 12
        b = packed & 4095
        col = v - TAIL_C0
        lo = plsc.load_gather(
            blkt, [iota, jnp.full((LANES,), 0, jnp.int32) + col]
        )
        hi = plsc.load_gather(
            blkt, [iota + LANES, jnp.full((LANES,), 0, jnp.int32) + col]
        )
        return fire_row(lo, hi, b, ft, fc3)

    fc2 = lax.fori_loop(s0, e0, tail_entry, fc, unroll=False)

    remaining = jnp.minimum(fc2, RING)

    def drain(j, c):
        @pl.when(j < remaining)
        def _():
            pltpu.make_async_copy(
                ring.at[0], out_hbm.at[pl.ds(0, DIM)], sem_out
            ).wait()
        return c

    lax.fori_loop(0, RING, drain, 0, unroll=False)


@jax.jit
def _emb(x_flat, tab2):
    mesh = plsc.VectorSubcoreMesh(core_axis_name="c", subcore_axis_name="s")

    bucket = functools.partial(
        pl.kernel,
        out_type=(
            jax.ShapeDtypeStruct((NUM_FEATURES * BATCH,), jnp.int32),
            jax.ShapeDtypeStruct((NUM_FEATURES * NBPAD,), jnp.int32),
        ),
        mesh=mesh,
        scratch_types=[
            pltpu.VMEM((BATCH,), jnp.int32),
            pltpu.VMEM((NBPAD * LANES,), jnp.int32),
            pltpu.VMEM((BATCH,), jnp.int32),
            pltpu.VMEM((NBPAD,), jnp.int32),
        ],
        compiler_params=pltpu.CompilerParams(needs_layout_passes=False),
    )(_bucket_kernel)
    sorted_a, starts_a = bucket(x_flat)

    sweep = functools.partial(
        pl.kernel,
        out_type=jax.ShapeDtypeStruct((BATCH * NUM_FEATURES * DIM,),
                                      jnp.float32),
        mesh=mesh,
        scratch_types=[
            pltpu.VMEM((NUM_FEATURES * NBPAD + LANES,), jnp.int32),
            pltpu.VMEM((2, BATCH + LANES), jnp.int32),
            pltpu.VMEM((NBUF, DIM, CW), jnp.float32),
            pltpu.VMEM((DIM, TAIL_CW), jnp.float32),
            pltpu.VMEM((RING, DIM), jnp.float32),
            pltpu.SemaphoreType.DMA,
            pltpu.SemaphoreType.DMA,
            pltpu.SemaphoreType.DMA,
        ],
        compiler_params=pltpu.CompilerParams(
            use_tc_tiling_on_sc=True, needs_layout_passes=False
        ),
    )(_sweep_kernel)
    return sweep(sorted_a, starts_a, tab2)


def kernel(x, tables):
    x_flat = x.astype(jnp.int32).T.reshape(NUM_FEATURES * BATCH)
    tab2 = jnp.transpose(tables, (0, 2, 1)).reshape(NUM_FEATURES * DIM, VOCAB)
    out = _emb(x_flat, tab2)
    return out.reshape(BATCH, NUM_FEATURES * DIM)


# speedup vs baseline: 5.3546x; 1.0127x over previous
import functools

import jax
import jax.numpy as jnp
from jax import lax
from jax.experimental import pallas as pl
from jax.experimental.pallas import tpu as pltpu
from jax.experimental.pallas import tpu_sc as plsc

NUM_FEATURES = 26
VOCAB = 100000
DIM = 32
BATCH = 4096

NUM_WORKERS = 32
LANES = 16
NB_FULL = VOCAB // 128
NBUCKETS = NB_FULL + 1
NBPAD = 784
TAIL_V0 = NB_FULL * 128
TAIL_W = VOCAB - TAIL_V0
CHUNK_B = 5
CW = CHUNK_B * 128
NCHUNKS = 156
CHUNKS_PER_W = 5
NBUF = 4
TAIL_C0 = NCHUNKS * CW
TAIL_CW = VOCAB - TAIL_C0
RING = 64


def _bucket_kernel(x_hbm, sorted_hbm, starts_hbm, xv, cur, sorted_v,
                   starts_v):
    wid = lax.axis_index("s") * 2 + lax.axis_index("c")

    @pl.when(wid < NUM_FEATURES)
    def _():
        f = wid
        pltpu.sync_copy(x_hbm.at[pl.ds(f * BATCH, BATCH)], xv)

        iota = lax.iota(jnp.int32, LANES)
        zeros = jnp.zeros((LANES,), jnp.int32)
        ones = jnp.ones((LANES,), jnp.int32)

        def zero_chunk(j, c):
            cur[pl.ds(j * LANES, LANES)] = zeros
            return c

        lax.fori_loop(0, NBPAD, zero_chunk, 0, unroll=False)

        def hist(i, c):
            v = xv[pl.ds(i * LANES, LANES)]
            addr = (v >> 7) * LANES + iota
            plsc.addupdate_scatter(cur, [addr], ones)
            return c

        lax.fori_loop(0, BATCH // LANES, hist, 0, unroll=False)

        def scan(j, run):
            c = cur[pl.ds(j * LANES, LANES)]
            s = plsc.cumsum(c)
            cur[pl.ds(j * LANES, LANES)] = s - c + run
            return run + jnp.sum(c)

        lax.fori_loop(0, NBPAD, scan, jnp.int32(0), unroll=False)

        def bounds(j, c):
            starts_v[pl.ds(j * LANES, LANES)] = plsc.load_gather(
                cur, [(j * LANES + iota) * LANES]
            )
            return c

        lax.fori_loop(0, NBPAD // LANES, bounds, 0, unroll=False)

        def permute(i, c):
            v = xv[pl.ds(i * LANES, LANES)]
            addr = (v >> 7) * LANES + iota
            pos = plsc.load_gather(cur, [addr])
            plsc.store_scatter(cur, [addr], pos + 1)
            packed = (v << 12) + (i * LANES + iota)
            plsc.store_scatter(sorted_v, [pos], packed)
            return c

        lax.fori_loop(0, BATCH // LANES, permute, 0, unroll=False)

        pltpu.sync_copy(sorted_v, sorted_hbm.at[pl.ds(f * BATCH, BATCH)])
        pltpu.sync_copy(starts_v, starts_hbm.at[pl.ds(f * NBPAD, NBPAD)])


def _sget(ref, idx):
    return ref[pl.ds(idx, LANES)][0]


def _sweep_kernel(sorted_hbm, starts_hbm, tab_hbm, out_hbm, starts_v, sv,
                  blk, blkt, ring, sem_blk, sem_out, sem_sv):
    wid = lax.axis_index("s") * 2 + lax.axis_index("c")
    iota = lax.iota(jnp.int32, LANES)
    ntasks = NUM_FEATURES * CHUNKS_PER_W

    pltpu.sync_copy(
        starts_hbm, starts_v.at[pl.ds(0, NUM_FEATURES * NBPAD)]
    )

    def fire_row(src_lo, src_hi, b, f, fc):
        slot = fc % RING

        @pl.when(fc >= RING)
        def _():
            pltpu.make_async_copy(
                ring.at[0], out_hbm.at[pl.ds(0, DIM)], sem_out
            ).wait()

        ring[slot, pl.ds(0, LANES)] = src_lo
        ring[slot, pl.ds(LANES, LANES)] = src_hi
        pltpu.async_copy(
            ring.at[slot], out_hbm.at[pl.ds(b * 832 + f * DIM, DIM)], sem_out
        )
        return fc + 1

    def chunk_of(t):
        f = t // CHUNKS_PER_W
        c = t % CHUNKS_PER_W
        ch = wid * CHUNKS_PER_W + c
        return f, ch

    def issue_read(t):
        f, ch = chunk_of(t)

        @pl.when(jnp.logical_and(t < ntasks, ch < NCHUNKS))
        def _():
            pltpu.async_copy(
                tab_hbm.at[pl.ds(f * DIM, DIM), pl.ds(ch * CW, CW)],
                blk.at[t % NBUF],
                sem_blk,
            )

    def issue_sv(f):
        @pl.when(f < NUM_FEATURES)
        def _():
            pltpu.async_copy(
                sorted_hbm.at[pl.ds(f * BATCH, BATCH)],
                sv.at[f % 2, pl.ds(0, BATCH)],
                sem_sv,
            )

    def wait_sv():
        pltpu.make_async_copy(
            sorted_hbm.at[pl.ds(0, BATCH)], sv.at[0, pl.ds(0, BATCH)],
            sem_sv,
        ).wait()

    issue_sv(0)
    issue_sv(1)
    wait_sv()
    for t in range(NBUF):
        issue_read(t)

    def do_task(t, fc):
        f, ch = chunk_of(t)
        valid = ch < NCHUNKS

        @pl.when(jnp.logical_and(t % CHUNKS_PER_W == 0, t > 0))
        def _():
            wait_sv()
            issue_sv(f + 1)

        @pl.when(valid)
        def _():
            pltpu.make_async_copy(
                tab_hbm.at[pl.ds(0, DIM), pl.ds(0, CW)], blk.at[t % NBUF],
                sem_blk,
            ).wait()

        vb_first = ch * CHUNK_B
        s0 = _sget(starts_v, f * NBPAD + vb_first)
        e0 = _sget(starts_v, f * NBPAD + vb_first + CHUNK_B)
        s0 = jnp.where(valid, s0, 0)
        e0 = jnp.where(valid, e0, 0)

        def do_entry(e, fc3):
            packed = sv[f % 2, pl.ds(e, LANES)][0]
            v = packed >> 12
            b = packed & 4095
            col = v - vb_first * 128
            lo = plsc.load_gather(
                blk,
                [jnp.full((LANES,), t % NBUF, jnp.int32), iota,
                 jnp.full((LANES,), 0, jnp.int32) + col],
            )
            hi = plsc.load_gather(
                blk,
                [jnp.full((LANES,), t % NBUF, jnp.int32), iota + LANES,
                 jnp.full((LANES,), 0, jnp.int32) + col],
            )
            return fire_row(lo, hi, b, f, fc3)

        fc2 = lax.fori_loop(s0, e0, do_entry, fc, unroll=False)
        issue_read(t + NBUF)
        return fc2

    fc = lax.fori_loop(0, ntasks, do_task, jnp.int32(0), unroll=False)

    has_tail = wid < NUM_FEATURES
    ft = jnp.minimum(wid, NUM_FEATURES - 1)

    @pl.when(has_tail)
    def _():
        pltpu.sync_copy(sorted_hbm.at[pl.ds(ft * BATCH, BATCH)],
                        sv.at[0, pl.ds(0, BATCH)])
        pltpu.sync_copy(
            tab_hbm.at[pl.ds(ft * DIM, DIM), pl.ds(TAIL_C0, TAIL_CW)], blkt
        )

    s0 = jnp.where(has_tail, _sget(starts_v, ft * NBPAD + NCHUNKS * CHUNK_B),
                   0)
    e0 = jnp.where(has_tail,
                   _sget(starts_v, ft * NBPAD + NCHUNKS * CHUNK_B + 2), 0)

    def tail_entry(e, fc3):
        packed = sv[0, pl.ds(e, LANES)][0]
        v = packed >> 12
        b = packed & 4095
        col = v - TAIL_C0
        lo = plsc.load_gather(
            blkt, [iota, jnp.full((LANES,), 0, jnp.int32) + col]
        )
        hi = plsc.load_gather(
            blkt, [iota + LANES, jnp.full((LANES,), 0, jnp.int32) + col]
        )
        return fire_row(lo, hi, b, ft, fc3)

    fc2 = lax.fori_loop(s0, e0, tail_entry, fc, unroll=False)

    remaining = jnp.minimum(fc2, RING)

    def drain(j, c):
        @pl.when(j < remaining)
        def _():
            pltpu.make_async_copy(
                ring.at[0], out_hbm.at[pl.ds(0, DIM)], sem_out
            ).wait()
        return c

    lax.fori_loop(0, RING, drain, 0, unroll=False)


@jax.jit
def _emb(x_flat, tab2):
    mesh = plsc.VectorSubcoreMesh(core_axis_name="c", subcore_axis_name="s")

    bucket = functools.partial(
        pl.kernel,
        out_type=(
            jax.ShapeDtypeStruct((NUM_FEATURES * BATCH,), jnp.int32),
            jax.ShapeDtypeStruct((NUM_FEATURES * NBPAD,), jnp.int32),
        ),
        mesh=mesh,
        scratch_types=[
            pltpu.VMEM((BATCH,), jnp.int32),
            pltpu.VMEM((NBPAD * LANES,), jnp.int32),
            pltpu.VMEM((BATCH,), jnp.int32),
            pltpu.VMEM((NBPAD,), jnp.int32),
        ],
        compiler_params=pltpu.CompilerParams(needs_layout_passes=False),
    )(_bucket_kernel)
    sorted_a, starts_a = bucket(x_flat)

    sweep = functools.partial(
        pl.kernel,
        out_type=jax.ShapeDtypeStruct((BATCH * NUM_FEATURES * DIM,),
                                      jnp.float32),
        mesh=mesh,
        scratch_types=[
            pltpu.VMEM((NUM_FEATURES * NBPAD + LANES,), jnp.int32),
            pltpu.VMEM((2, BATCH + LANES), jnp.int32),
            pltpu.VMEM((NBUF, DIM, CW), jnp.float32),
            pltpu.VMEM((DIM, TAIL_CW), jnp.float32),
            pltpu.VMEM((RING, DIM), jnp.float32),
            pltpu.SemaphoreType.DMA,
            pltpu.SemaphoreType.DMA,
            pltpu.SemaphoreType.DMA,
        ],
        compiler_params=pltpu.CompilerParams(
            use_tc_tiling_on_sc=True, needs_layout_passes=False
        ),
    )(_sweep_kernel)
    return sweep(sorted_a, starts_a, tab2)


def kernel(x, tables):
    x_flat = x.astype(jnp.int32).T.reshape(NUM_FEATURES * BATCH)
    tab2 = jnp.transpose(tables, (0, 2, 1)).reshape(NUM_FEATURES * DIM, VOCAB)
    out = _emb(x_flat, tab2)
    return out.reshape(BATCH, NUM_FEATURES * DIM)


# speedup vs baseline: 5.3572x; 1.0005x over previous
import functools

import jax
import jax.numpy as jnp
from jax import lax
from jax.experimental import pallas as pl
from jax.experimental.pallas import tpu as pltpu
from jax.experimental.pallas import tpu_sc as plsc

NUM_FEATURES = 26
VOCAB = 100000
DIM = 32
BATCH = 4096

LANES = 16
NBPAD = 784
CHUNK_B = 5
CW = CHUNK_B * 128
NCHUNKS = 156
CHUNKS_PER_W = 5
NBUF = 4
TAIL_C0 = NCHUNKS * CW
TAIL_CW = VOCAB - TAIL_C0
RING = 64


def _bucket_kernel(x_hbm, sorted_hbm, starts_hbm, xv, cur, sorted_v,
                   starts_v):
    wid = lax.axis_index("s") * 2 + lax.axis_index("c")

    @pl.when(wid < NUM_FEATURES)
    def _():
        f = wid
        pltpu.sync_copy(x_hbm.at[pl.ds(f * BATCH, BATCH)], xv)

        iota = lax.iota(jnp.int32, LANES)
        zeros = jnp.zeros((LANES,), jnp.int32)
        ones = jnp.ones((LANES,), jnp.int32)

        def zero_chunk(j, c):
            cur[pl.ds(j * LANES, LANES)] = zeros
            return c

        lax.fori_loop(0, NBPAD, zero_chunk, 0, unroll=False)

        def hist(i, c):
            v = xv[pl.ds(i * LANES, LANES)]
            addr = (v >> 7) * LANES + iota
            plsc.addupdate_scatter(cur, [addr], ones)
            return c

        lax.fori_loop(0, BATCH // LANES, hist, 0, unroll=False)

        def scan(j, run):
            c = cur[pl.ds(j * LANES, LANES)]
            s = plsc.cumsum(c)
            cur[pl.ds(j * LANES, LANES)] = s - c + run
            return run + jnp.sum(c)

        lax.fori_loop(0, NBPAD, scan, jnp.int32(0), unroll=False)

        def bounds(j, c):
            starts_v[pl.ds(j * LANES, LANES)] = plsc.load_gather(
                cur, [(j * LANES + iota) * LANES]
            )
            return c

        lax.fori_loop(0, NBPAD // LANES, bounds, 0, unroll=False)

        def permute(i, c):
            v = xv[pl.ds(i * LANES, LANES)]
            addr = (v >> 7) * LANES + iota
            pos = plsc.load_gather(cur, [addr])
            plsc.store_scatter(cur, [addr], pos + 1)
            packed = (v << 12) + (i * LANES + iota)
            plsc.store_scatter(sorted_v, [pos], packed)
            return c

        lax.fori_loop(0, BATCH // LANES, permute, 0, unroll=False)

        pltpu.sync_copy(sorted_v, sorted_hbm.at[pl.ds(f * BATCH, BATCH)])
        pltpu.sync_copy(starts_v, starts_hbm.at[pl.ds(f * NBPAD, NBPAD)])


def _sget(ref, idx):
    return ref[pl.ds(idx, LANES)][0]


def _sweep_kernel(sorted_hbm, starts_hbm, tab_hbm, out_hbm, starts_v, sv,
                  blk, blkt, ring, sem_blk, sem_out, sem_sv):
    wid = lax.axis_index("s") * 2 + lax.axis_index("c")
    iota = lax.iota(jnp.int32, LANES)
    ntasks = NUM_FEATURES * CHUNKS_PER_W

    pltpu.sync_copy(
        starts_hbm, starts_v.at[pl.ds(0, NUM_FEATURES * NBPAD)]
    )

    def fire_row(src_lo, src_hi, b, f, fc):
        slot = fc % RING

        @pl.when(fc >= RING)
        def _():
            pltpu.make_async_copy(
                ring.at[0], out_hbm.at[pl.ds(0, DIM)], sem_out
            ).wait()

        ring[slot, pl.ds(0, LANES)] = src_lo
        ring[slot, pl.ds(LANES, LANES)] = src_hi
        pltpu.async_copy(
            ring.at[slot], out_hbm.at[pl.ds(b * 832 + f * DIM, DIM)], sem_out
        )
        return fc + 1

    def chunk_of(t):
        f = t // CHUNKS_PER_W
        c = t % CHUNKS_PER_W
        ch = wid * CHUNKS_PER_W + c
        return f, ch

    def issue_read(t):
        f, ch = chunk_of(t)

        @pl.when(jnp.logical_and(t < ntasks, ch < NCHUNKS))
        def _():
            pltpu.async_copy(
                tab_hbm.at[pl.ds(f * DIM, DIM), pl.ds(ch * CW, CW)],
                blk.at[t % NBUF],
                sem_blk,
            )

    def issue_sv(f):
        @pl.when(f < NUM_FEATURES)
        def _():
            pltpu.async_copy(
                sorted_hbm.at[pl.ds(f * BATCH, BATCH)],
                sv.at[f % 2, pl.ds(0, BATCH)],
                sem_sv,
            )

    def wait_sv():
        pltpu.make_async_copy(
            sorted_hbm.at[pl.ds(0, BATCH)], sv.at[0, pl.ds(0, BATCH)],
            sem_sv,
        ).wait()

    issue_sv(0)
    issue_sv(1)
    wait_sv()
    for t in range(NBUF):
        issue_read(t)

    def do_task(t, fc):
        f, ch = chunk_of(t)
        valid = ch < NCHUNKS

        @pl.when(jnp.logical_and(t % CHUNKS_PER_W == 0, t > 0))
        def _():
            wait_sv()
            issue_sv(f + 1)

        @pl.when(valid)
        def _():
            pltpu.make_async_copy(
                tab_hbm.at[pl.ds(0, DIM), pl.ds(0, CW)], blk.at[t % NBUF],
                sem_blk,
            ).wait()

        vb_first = ch * CHUNK_B
        s0 = _sget(starts_v, f * NBPAD + vb_first)
        e0 = _sget(starts_v, f * NBPAD + vb_first + CHUNK_B)
        s0 = jnp.where(valid, s0, 0)
        e0 = jnp.where(valid, e0, 0)

        def do_entry(e, fc3):
            packed = sv[f % 2, pl.ds(e, LANES)][0]
            v = packed >> 12
            b = packed & 4095
            col = v - vb_first * 128
            lo = plsc.load_gather(
                blk,
                [jnp.full((LANES,), t % NBUF, jnp.int32), iota,
                 jnp.full((LANES,), 0, jnp.int32) + col],
            )
            hi = plsc.load_gather(
                blk,
                [jnp.full((LANES,), t % NBUF, jnp.int32), iota + LANES,
                 jnp.full((LANES,), 0, jnp.int32) + col],
            )
            return fire_row(lo, hi, b, f, fc3)

        fc2 = lax.fori_loop(s0, e0, do_entry, fc, unroll=False)
        issue_read(t + NBUF)
        return fc2

    fc = lax.fori_loop(0, ntasks, do_task, jnp.int32(0), unroll=False)

    has_tail = wid < NUM_FEATURES
    ft = jnp.minimum(wid, NUM_FEATURES - 1)

    @pl.when(has_tail)
    def _():
        pltpu.sync_copy(sorted_hbm.at[pl.ds(ft * BATCH, BATCH)],
                        sv.at[0, pl.ds(0, BATCH)])
        pltpu.sync_copy(
            tab_hbm.at[pl.ds(ft * DIM, DIM), pl.ds(TAIL_C0, TAIL_CW)], blkt
        )

    s0 = jnp.where(has_tail, _sget(starts_v, ft * NBPAD + NCHUNKS * CHUNK_B),
                   0)
    e0 = jnp.where(has_tail,
                   _sget(starts_v, ft * NBPAD + NCHUNKS * CHUNK_B + 2), 0)

    def tail_entry(e, fc3):
        packed = sv[0, pl.ds(e, LANES)][0]
        v = packed >> 12
        b = packed & 4095
        col = v - TAIL_C0
        lo = plsc.load_gather(
            blkt, [iota, jnp.full((LANES,), 0, jnp.int32) + col]
        )
        hi = plsc.load_gather(
            blkt, [iota + LANES, jnp.full((LANES,), 0, jnp.int32) + col]
        )
        return fire_row(lo, hi, b, ft, fc3)

    fc2 = lax.fori_loop(s0, e0, tail_entry, fc, unroll=False)

    remaining = jnp.minimum(fc2, RING)

    def drain(j, c):
        @pl.when(j < remaining)
        def _():
            pltpu.make_async_copy(
                ring.at[0], out_hbm.at[pl.ds(0, DIM)], sem_out
            ).wait()
        return c

    lax.fori_loop(0, RING, drain, 0, unroll=False)


@jax.jit
def _emb(x_flat, tab2):
    mesh = plsc.VectorSubcoreMesh(core_axis_name="c", subcore_axis_name="s")

    bucket = functools.partial(
        pl.kernel,
        out_type=(
            jax.ShapeDtypeStruct((NUM_FEATURES * BATCH,), jnp.int32),
            jax.ShapeDtypeStruct((NUM_FEATURES * NBPAD,), jnp.int32),
        ),
        mesh=mesh,
        scratch_types=[
            pltpu.VMEM((BATCH,), jnp.int32),
            pltpu.VMEM((NBPAD * LANES,), jnp.int32),
            pltpu.VMEM((BATCH,), jnp.int32),
            pltpu.VMEM((NBPAD,), jnp.int32),
        ],
        compiler_params=pltpu.CompilerParams(needs_layout_passes=False),
    )(_bucket_kernel)
    sorted_a, starts_a = bucket(x_flat)

    sweep = functools.partial(
        pl.kernel,
        out_type=jax.ShapeDtypeStruct((BATCH * NUM_FEATURES * DIM,),
                                      jnp.float32),
        mesh=mesh,
        scratch_types=[
            pltpu.VMEM((NUM_FEATURES * NBPAD + LANES,), jnp.int32),
            pltpu.VMEM((2, BATCH + LANES), jnp.int32),
            pltpu.VMEM((NBUF, DIM, CW), jnp.float32),
            pltpu.VMEM((DIM, TAIL_CW), jnp.float32),
            pltpu.VMEM((RING, DIM), jnp.float32),
            pltpu.SemaphoreType.DMA,
            pltpu.SemaphoreType.DMA,
            pltpu.SemaphoreType.DMA,
        ],
        compiler_params=pltpu.CompilerParams(
            use_tc_tiling_on_sc=True, needs_layout_passes=False
        ),
    )(_sweep_kernel)
    return sweep(sorted_a, starts_a, tab2)


def kernel(x, tables):
    x_flat = x.astype(jnp.int32).T.reshape(NUM_FEATURES * BATCH)
    tab2 = jnp.transpose(tables, (0, 2, 1)).reshape(NUM_FEATURES * DIM, VOCAB)
    out = _emb(x_flat, tab2)
    return out.reshape(BATCH, NUM_FEATURES * DIM)
